# baseline (speedup 1.0000x reference)
import jax
import jax.numpy as jnp
import numpy as np
from jax.experimental import pallas as pl
from jax.experimental.pallas import tpu as pltpu

_EPS = 0.1
_NSLOT = 4
_NSPLIT = 4


def _splits(tr):
    per = (tr // _NSPLIT) // 8 * 8
    starts = [8 + i * per for i in range(_NSPLIT)]
    sizes = [per] * (_NSPLIT - 1) + [tr - per * (_NSPLIT - 1)]
    return starts, sizes


def _fill_kernel(x_ref, o_hbm, tmpl, head, tsems, hsems):
    i = pl.program_id(0)
    n = pl.num_programs(0)
    p = tmpl.shape[1]
    tr = tmpl.shape[0]
    starts, sizes = _splits(tr)

    @pl.when(i == 0)
    def _init():
        r = jax.lax.broadcasted_iota(jnp.int32, (tr, p), 0)
        c = jax.lax.broadcasted_iota(jnp.int32, (tr, p), 1)
        tmpl[...] = jnp.where(c == r + 7, _EPS, 0.0).astype(tmpl.dtype)
        hr = jax.lax.broadcasted_iota(jnp.int32, (8, p), 0)
        hc = jax.lax.broadcasted_iota(jnp.int32, (8, p), 1)
        hbase = jnp.where(hr == hc + 1, _EPS, 0.0).astype(head.dtype)
        for s in range(_NSLOT):
            head[s] = hbase

    slot = jax.lax.rem(i, _NSLOT)

    def _tmpl_dmas(it, sl, start):
        for q in range(_NSPLIT):
            cp = pltpu.make_async_copy(
                tmpl.at[pl.ds(starts[q] - 8, sizes[q]), :],
                o_hbm.at[it, pl.ds(starts[q], sizes[q]), :],
                tsems.at[sl, q],
            )
            if start:
                cp.start()
            else:
                cp.wait()

    def _head_dma(it, sl, start):
        cp = pltpu.make_async_copy(
            head.at[sl], o_hbm.at[it, pl.ds(0, 8), :], hsems.at[sl]
        )
        if start:
            cp.start()
        else:
            cp.wait()

    @pl.when(i >= _NSLOT)
    def _wait_prev():
        _tmpl_dmas(i - _NSLOT, slot, start=False)
        _head_dma(i - _NSLOT, slot, start=False)

    head[slot, pl.ds(0, 1), :] = x_ref[0]
    _head_dma(i, slot, start=True)
    _tmpl_dmas(i, slot, start=True)

    @pl.when(i == n - 1)
    def _drain():
        for j in range(_NSLOT):
            it = n - _NSLOT + j
            _tmpl_dmas(it, it % _NSLOT, start=False)
            _head_dma(it, it % _NSLOT, start=False)


_NCHUNK = 4


def kernel(x):
    n = x.shape[0]
    inner = x.shape[1:]
    p = int(np.prod(inner))
    xf = x.reshape(n, 1, p)
    nb = n // _NCHUNK
    fill = pl.pallas_call(
        _fill_kernel,
        grid=(nb,),
        in_specs=[pl.BlockSpec((1, 1, p), lambda i: (i, 0, 0))],
        out_specs=pl.BlockSpec(memory_space=pl.ANY),
        out_shape=jax.ShapeDtypeStruct((nb, 1 + p, p), x.dtype),
        scratch_shapes=[
            pltpu.VMEM((p - 7, p), x.dtype),
            pltpu.VMEM((_NSLOT, 8, p), x.dtype),
            pltpu.SemaphoreType.DMA((_NSLOT, _NSPLIT)),
            pltpu.SemaphoreType.DMA((_NSLOT,)),
        ],
        compiler_params=pltpu.CompilerParams(
            dimension_semantics=("arbitrary",),
        ),
    )
    parts = [
        fill(xf[g * nb : (g + 1) * nb]).reshape((nb, 1 + p) + tuple(inner))
        for g in range(_NCHUNK)
    ]
    return jnp.concatenate(parts, axis=0)


# speedup vs baseline: 1.2829x; 1.2829x over previous
import functools

import jax
import jax.numpy as jnp
import numpy as np
from jax import lax
from jax.experimental import pallas as pl
from jax.experimental.pallas import tpu as pltpu
from jax.experimental.pallas import tpu_sc as plsc

_EPS = 0.1
_CH = 56


def _to_z_sc(n, p, x_hbm, o_hbm, bufs, tail, xblk, sems, tsem):
    info = plsc.get_sparse_core_info()
    nc, ns = info.num_cores, info.num_subcores
    nw = nc * ns
    nch = (1 + p) // _CH
    per_w = n // nw

    wid = lax.axis_index("s") * nc + lax.axis_index("c")
    lanes = jnp.arange(16, dtype=jnp.int32)
    zeros16 = jnp.zeros((16,), jnp.float32)

    def _onehot(col):
        return jnp.where(lanes == col % 16, _EPS, 0.0).astype(jnp.float32)

    def _grp(col):
        return (col // 16) * 16

    def _zero_row(r, _):
        def _zero_seg(q, _):
            for b in range(2):
                bufs[b, r, pl.ds(q * 16, 16)] = zeros16
            return 0

        return lax.fori_loop(0, p // 16, _zero_seg, 0)

    lax.fori_loop(0, _CH, _zero_row, 0)

    def _zero_tail(q, _):
        tail[0, pl.ds(q * 16, 16)] = zeros16
        return 0

    lax.fori_loop(0, p // 16, _zero_tail, 0)
    tail[0, pl.ds(_grp(p - 1), 16)] = _onehot(p - 1)

    def _chunk_dmas(b, c, start):
        for s in range(per_w):
            batch = wid * per_w + s
            cp = pltpu.make_async_copy(
                bufs.at[b],
                o_hbm.at[batch, pl.ds(c * _CH, _CH), :],
                sems.at[b],
            )
            if start:
                cp.start()
            else:
                cp.wait()

    blk0 = (wid * per_w // 8) * 8
    pltpu.sync_copy(x_hbm.at[pl.ds(blk0, 8), :], xblk)

    def _set_c0(j, _):
        bufs[0, j, pl.ds(_grp(j - 1), 16)] = _onehot(j - 1)
        return 0

    lax.fori_loop(1, _CH, _set_c0, 0)
    for s in range(per_w):
        batch = wid * per_w + s

        def _copy_x(q, _):
            bufs[0, 0, pl.ds(q * 16, 16)] = xblk[
                wid * per_w + s - blk0, pl.ds(q * 16, 16)
            ]
            return 0

        lax.fori_loop(0, p // 16, _copy_x, 0)
        pltpu.make_async_copy(
            bufs.at[0], o_hbm.at[batch, pl.ds(0, _CH), :], sems.at[0]
        ).start()
        pltpu.make_async_copy(
            bufs.at[0], o_hbm.at[batch, pl.ds(0, _CH), :], sems.at[0]
        ).wait()
        pltpu.make_async_copy(
            tail, o_hbm.at[batch, pl.ds(nch * _CH, 1), :], tsem
        ).start()

    def _clr_c0(j, _):
        bufs[0, j, pl.ds(_grp(j - 1), 16)] = zeros16
        return 0

    lax.fori_loop(1, _CH, _clr_c0, 0)

    def _clr_x(q, _):
        bufs[0, 0, pl.ds(q * 16, 16)] = zeros16
        return 0

    lax.fori_loop(0, p // 16, _clr_x, 0)

    def _do_chunk(b, c):
        @pl.when(c > 2)
        def _wait_prev():
            _chunk_dmas(b, c - 2, start=False)

        def _edit_row(j, _):
            col_new = c * _CH + j - 1
            col_old = col_new - 2 * _CH

            @pl.when(col_old >= 0)
            def _clr():
                bufs[b, j, pl.ds(_grp(col_old), 16)] = zeros16

            bufs[b, j, pl.ds(_grp(col_new), 16)] = _onehot(col_new)
            return 0

        lax.fori_loop(0, _CH, _edit_row, 0)
        _chunk_dmas(b, c, start=True)

    def _pair(t, _):
        for b in range(2):
            _do_chunk(b, 2 * t + b + 1)
        return 0

    lax.fori_loop(0, (nch - 2) // 2, _pair, 0)
    _do_chunk(0, jnp.int32(nch - 1))

    _chunk_dmas(1, nch - 2, start=False)
    _chunk_dmas(0, nch - 1, start=False)
    for s in range(per_w):
        batch = wid * per_w + s
        pltpu.make_async_copy(
            tail, o_hbm.at[batch, pl.ds(nch * _CH, 1), :], tsem
        ).wait()


def _head_patch_kernel(x_ref, flat_ref, o_hbm, head, sem):
    del flat_ref
    p = head.shape[1]
    hr = jax.lax.broadcasted_iota(jnp.int32, (8, p), 0)
    hc = jax.lax.broadcasted_iota(jnp.int32, (8, p), 1)
    head[...] = jnp.where(hr == hc + 1, _EPS, 0.0).astype(head.dtype)
    head[pl.ds(0, 1), :] = x_ref[0]
    cp = pltpu.make_async_copy(head, o_hbm.at[0, pl.ds(0, 8), :], sem)
    cp.start()
    cp.wait()


def kernel(x):
    n = x.shape[0]
    inner = x.shape[1:]
    p = int(np.prod(inner))
    xf = x.reshape(n, p)
    mesh = plsc.VectorSubcoreMesh(core_axis_name="c", subcore_axis_name="s")
    flat = pl.kernel(
        functools.partial(_to_z_sc, n, p),
        out_type=jax.ShapeDtypeStruct((n, 1 + p, p), x.dtype),
        mesh=mesh,
        scratch_types=[
            pltpu.VMEM((2, _CH, p), jnp.float32),
            pltpu.VMEM((1, p), jnp.float32),
            pltpu.VMEM((8, p), jnp.float32),
            pltpu.SemaphoreType.DMA((2,)),
            pltpu.SemaphoreType.DMA,
        ],
        compiler_params=pltpu.CompilerParams(use_tc_tiling_on_sc=True),
    )(xf)
    out = pl.pallas_call(
        _head_patch_kernel,
        grid=(1,),
        in_specs=[
            pl.BlockSpec((1, 1, p), lambda i: (0, 0, 0)),
            pl.BlockSpec(memory_space=pl.ANY),
        ],
        out_specs=pl.BlockSpec(memory_space=pl.ANY),
        out_shape=jax.ShapeDtypeStruct((n, 1 + p, p), x.dtype),
        input_output_aliases={1: 0},
        scratch_shapes=[
            pltpu.VMEM((8, p), x.dtype),
            pltpu.SemaphoreType.DMA,
        ],
        compiler_params=pltpu.CompilerParams(
            dimension_semantics=("arbitrary",),
        ),
    )(x.reshape(n, 1, p), flat)
    return out.reshape((n, 1 + p) + tuple(inner))


# speedup vs baseline: 1.2900x; 1.0055x over previous
import functools

import jax
import jax.numpy as jnp
import numpy as np
from jax import lax
from jax.experimental import pallas as pl
from jax.experimental.pallas import tpu as pltpu
from jax.experimental.pallas import tpu_sc as plsc

_EPS = 0.1
_CH = 56


def _to_z_sc(n, p, x_hbm, o_hbm, bufs, tail, xblk, sems, tsem):
    info = plsc.get_sparse_core_info()
    nc, ns = info.num_cores, info.num_subcores
    nw = nc * ns
    nch = (1 + p) // _CH
    per_w = n // nw

    wid = lax.axis_index("s") * nc + lax.axis_index("c")
    lanes = jnp.arange(16, dtype=jnp.int32)
    zeros16 = jnp.zeros((16,), jnp.float32)

    def _onehot(col):
        return jnp.where(lanes == col % 16, _EPS, 0.0).astype(jnp.float32)

    def _grp(col):
        return (col // 16) * 16

    def _zero_row(r, _):
        def _zero_seg(q, _):
            for b in range(2):
                bufs[b, r, pl.ds(q * 16, 16)] = zeros16
            return 0

        return lax.fori_loop(0, p // 16, _zero_seg, 0)

    lax.fori_loop(0, _CH, _zero_row, 0)

    def _zero_tail(q, _):
        tail[0, pl.ds(q * 16, 16)] = zeros16
        return 0

    lax.fori_loop(0, p // 16, _zero_tail, 0)
    tail[0, pl.ds(_grp(p - 1), 16)] = _onehot(p - 1)

    def _chunk_dmas(b, c, start):
        for s in range(per_w):
            batch = wid * per_w + s
            cp = pltpu.make_async_copy(
                bufs.at[b],
                o_hbm.at[batch, pl.ds(c * _CH, _CH), :],
                sems.at[b],
            )
            if start:
                cp.start()
            else:
                cp.wait()

    blk0 = (wid * per_w // 8) * 8
    pltpu.sync_copy(x_hbm.at[pl.ds(blk0, 8), :], xblk)

    def _set_c0(j, _):
        bufs[0, j, pl.ds(_grp(j - 1), 16)] = _onehot(j - 1)
        return 0

    lax.fori_loop(1, _CH, _set_c0, 0)
    for s in range(per_w):
        batch = wid * per_w + s

        def _copy_x(q, _):
            bufs[0, 0, pl.ds(q * 16, 16)] = xblk[
                wid * per_w + s - blk0, pl.ds(q * 16, 16)
            ]
            return 0

        lax.fori_loop(0, p // 16, _copy_x, 0)
        pltpu.make_async_copy(
            bufs.at[0], o_hbm.at[batch, pl.ds(0, _CH), :], sems.at[0]
        ).start()
        pltpu.make_async_copy(
            bufs.at[0], o_hbm.at[batch, pl.ds(0, _CH), :], sems.at[0]
        ).wait()
        pltpu.make_async_copy(
            tail, o_hbm.at[batch, pl.ds(nch * _CH, 1), :], tsem
        ).start()

    def _clr_c0(j, _):
        bufs[0, j, pl.ds(_grp(j - 1), 16)] = zeros16
        return 0

    lax.fori_loop(1, _CH, _clr_c0, 0)

    def _clr_x(q, _):
        bufs[0, 0, pl.ds(q * 16, 16)] = zeros16
        return 0

    lax.fori_loop(0, p // 16, _clr_x, 0)

    def _do_chunk(b, c):
        @pl.when(c > 2)
        def _wait_prev():
            _chunk_dmas(b, c - 2, start=False)

        def _edit_row(j, _):
            col_new = c * _CH + j - 1
            col_old = col_new - 2 * _CH

            @pl.when(col_old >= 0)
            def _clr():
                bufs[b, j, pl.ds(_grp(col_old), 16)] = zeros16

            bufs[b, j, pl.ds(_grp(col_new), 16)] = _onehot(col_new)
            return 0

        lax.fori_loop(0, _CH, _edit_row, 0)
        _chunk_dmas(b, c, start=True)

    def _pair(t, _):
        for b in range(2):
            _do_chunk(b, 2 * t + b + 1)
        return 0

    lax.fori_loop(0, (nch - 2) // 2, _pair, 0)
    _do_chunk(0, jnp.int32(nch - 1))

    _chunk_dmas(1, nch - 2, start=False)
    _chunk_dmas(0, nch - 1, start=False)
    for s in range(per_w):
        batch = wid * per_w + s
        pltpu.make_async_copy(
            tail, o_hbm.at[batch, pl.ds(nch * _CH, 1), :], tsem
        ).wait()


def kernel(x):
    n = x.shape[0]
    inner = x.shape[1:]
    p = int(np.prod(inner))
    xf = x.reshape(n, p)
    mesh = plsc.VectorSubcoreMesh(core_axis_name="c", subcore_axis_name="s")
    out = pl.kernel(
        functools.partial(_to_z_sc, n, p),
        out_type=jax.ShapeDtypeStruct((n, 1 + p, p), x.dtype),
        mesh=mesh,
        scratch_types=[
            pltpu.VMEM((2, _CH, p), jnp.float32),
            pltpu.VMEM((1, p), jnp.float32),
            pltpu.VMEM((8, p), jnp.float32),
            pltpu.SemaphoreType.DMA((2,)),
            pltpu.SemaphoreType.DMA,
        ],
        compiler_params=pltpu.CompilerParams(use_tc_tiling_on_sc=True),
    )(xf)
    return out.reshape((n, 1 + p) + tuple(inner))
